# baseline (speedup 1.0000x reference)
import functools

import jax
import jax.numpy as jnp
from jax import lax
from jax.experimental import pallas as pl
from jax.experimental.pallas import tpu as pltpu
from jax.experimental.pallas import tpu_sc as plsc

BATCH = 16384
EMB_D = 64
NROWS = 1000000
NC = 2
NS = 16
NW = NC * NS
L = 16

NSLAB = 7813
LAST_SLAB = 7812
RANGE = 245
NGRP = 16
GS = 16
MYCAP = 704
GCAP = 144
SENT = 2048000


def _sc_gather_body(idx_hbm, tableT_hbm, out_hbm,
                    idx_v, my_r, my_pos, grp_r, grp_pos, hit_r, hit_pos,
                    slab_v, rows_c, pfx_v, wpos_s, ssem, wsem):
    wid = lax.axis_index("s") * NC + lax.axis_index("c")
    s0 = wid * RANGE
    iota = lax.iota(jnp.int32, L)
    sent_v = jnp.full((L,), SENT, jnp.int32)
    zero_v = jnp.zeros((L,), jnp.int32)

    def prefix16(x):
        for sh in (1, 2, 4, 8):
            pfx_v[pl.ds(0, L)] = zero_v
            plsc.store_scatter(pfx_v, [jnp.minimum(iota + sh, L - 1)], x,
                               mask=iota + sh < L)
            x = x + pfx_v[pl.ds(0, L)]
        return x

    for c in range(MYCAP // L):
        my_r[pl.ds(c * L, L)] = sent_v

    lo = s0 * 128
    hi = (s0 + RANGE) * 128
    ICH = 2048

    cnt0 = jnp.int32(0)
    for ch in range(BATCH // ICH):
        pltpu.sync_copy(idx_hbm.at[pl.ds(ch * ICH, ICH)], idx_v)

        def l0(k, cnt, _ch=ch):
            r = idx_v[pl.ds(k * L, L)]
            ge = ((r - lo) >> 31) + 1
            lt = -((r - hi) >> 31)
            cs = prefix16(ge * lt)
            m = ge * lt > 0
            plsc.store_scatter(my_r, [cnt + cs - 1], r, mask=m)
            plsc.store_scatter(my_pos, [cnt + cs - 1],
                              iota + _ch * ICH + k * L, mask=m)
            return cnt + cs[L - 1]

        cnt0 = lax.fori_loop(0, ICH // L, l0, cnt0)

    def group_body(g, hitcnt):
        g0 = s0 + g * GS
        for c in range(GCAP // L):
            grp_r[pl.ds(c * L, L)] = sent_v

        def l1(c, gcnt):
            r = my_r[pl.ds(c * L, L)]
            p = my_pos[pl.ds(c * L, L)]
            sl = r >> 7
            ge = ((sl - g0) >> 31) + 1
            lt = -((sl - (g0 + GS)) >> 31)
            cs = prefix16(ge * lt)
            m = ge * lt > 0
            plsc.store_scatter(grp_r, [gcnt + cs - 1], r, mask=m)
            plsc.store_scatter(grp_pos, [gcnt + cs - 1], p, mask=m)
            return gcnt + cs[L - 1]

        lax.fori_loop(0, MYCAP // L, l1, jnp.int32(0))

        def fetch(s, b):
            pltpu.async_copy(tableT_hbm.at[:, pl.ds(s * 128, 128)],
                             slab_v.at[b], ssem)

        @pl.when(g0 <= LAST_SLAB)
        def _():
            fetch(g0, 0)

        def slab_body(t, hc):
            s = g0 + t

            @pl.when(s <= LAST_SLAB)
            def _():
                pltpu.make_async_copy(
                    tableT_hbm.at[:, pl.ds(s * 128, 128)],
                    slab_v.at[t % 2], ssem).wait()

            @pl.when(jnp.logical_and(t + 1 < GS, s + 1 <= LAST_SLAB))
            def _():
                fetch(s + 1, (t + 1) % 2)

            slab = slab_v.at[t % 2]
            for c in range(GCAP // L):
                r = grp_r[pl.ds(c * L, L)]
                d = (r >> 7) - s
                x = 1 - jnp.minimum(jnp.abs(d), 1)
                cs = prefix16(x)
                m = x > 0
                pc = cs[L - 1]

                @pl.when(pc > 0)
                def _():
                    p = grp_pos[pl.ds(c * L, L)]
                    plsc.store_scatter(hit_r, [cs - 1], r, mask=m)
                    plsc.store_scatter(hit_pos, [cs - 1], p, mask=m)
                    hr = hit_r[pl.ds(0, L)]
                    hp = hit_pos[pl.ds(0, L)]
                    for l in range(L):
                        @pl.when(l < pc)
                        def _():
                            lane = hr[l] & 127
                            k = hc + l
                            wpos_s[k] = hp[l]
                            lane_v = iota * 0 + lane
                            for c4 in range(EMB_D // L):
                                v = plsc.load_gather(
                                    slab, [iota + c4 * L, lane_v])
                                rows_c[k, pl.ds(c4 * L, L)] = v

                hc = hc + pc
            return hc

        return lax.fori_loop(0, GS, slab_body, hitcnt)

    hitcnt = lax.fori_loop(0, NGRP, group_body, jnp.int32(0))

    def wr(k, _):
        pltpu.async_copy(rows_c.at[k], out_hbm.at[wpos_s[k]], wsem)
        return _

    lax.fori_loop(0, hitcnt, wr, 0)

    def drain(k, _):
        pltpu.make_async_copy(rows_c.at[0], out_hbm.at[0], wsem).wait()
        return _

    lax.fori_loop(0, hitcnt, drain, 0)


def _sc_gather(idx, tableT):
    mesh = plsc.VectorSubcoreMesh(core_axis_name="c", subcore_axis_name="s")
    k = functools.partial(
        pl.kernel,
        mesh=mesh,
        out_type=jax.ShapeDtypeStruct((BATCH, EMB_D), jnp.float32),
        scratch_types=[
            pltpu.VMEM((2048,), jnp.int32),
            pltpu.VMEM((MYCAP,), jnp.int32),
            pltpu.VMEM((MYCAP,), jnp.int32),
            pltpu.VMEM((GCAP,), jnp.int32),
            pltpu.VMEM((GCAP,), jnp.int32),
            pltpu.VMEM((L,), jnp.int32),
            pltpu.VMEM((L,), jnp.int32),
            pltpu.VMEM((2, EMB_D, 128), jnp.float32),
            pltpu.VMEM((MYCAP, EMB_D), jnp.float32),
            pltpu.VMEM((L,), jnp.int32),
            pltpu.SMEM((MYCAP,), jnp.int32),
            pltpu.SemaphoreType.DMA,
            pltpu.SemaphoreType.DMA,
        ],
        compiler_params=pltpu.CompilerParams(disable_bounds_checks=True,
                                             needs_layout_passes=False),
    )(_sc_gather_body)
    return k(idx, tableT)


def _mlp_body(x_ref, w1_ref, b1_ref, w2_ref, b2_ref, w3_ref, b3_ref, o_ref):
    x = x_ref[...]
    h = jnp.dot(x, w1_ref[...], preferred_element_type=jnp.float32)
    h = jnp.maximum(h + b1_ref[...], 0.0)
    h = jnp.dot(h, w2_ref[...], preferred_element_type=jnp.float32)
    h = jnp.maximum(h + b2_ref[...], 0.0)
    y = jnp.dot(h, w3_ref[...], preferred_element_type=jnp.float32)
    y = y + b3_ref[...]
    norm = jnp.sqrt(jnp.sum(y * y, axis=1, keepdims=True))
    o_ref[...] = y / jnp.maximum(norm, 1e-12)


def _mlp(x, W1, b1, W2, b2, W3, b3):
    blk = 2048
    grid = (BATCH // blk,)
    full = lambda shape: pl.BlockSpec(shape, lambda i: (0, 0))
    return pl.pallas_call(
        _mlp_body,
        grid=grid,
        in_specs=[
            pl.BlockSpec((blk, EMB_D), lambda i: (i, 0)),
            full(W1.shape), full(b1.shape), full(W2.shape),
            full(b2.shape), full(W3.shape), full(b3.shape),
        ],
        out_specs=pl.BlockSpec((blk, EMB_D), lambda i: (i, 0)),
        out_shape=jax.ShapeDtypeStruct((BATCH, EMB_D), jnp.float32),
    )(x, W1, b1, W2, b2, W3, b3)


def kernel(user_ids, emb_table, W1, b1, W2, b2, W3, b3):
    idx = user_ids.astype(jnp.int32)
    gathered = _sc_gather(idx, emb_table.T)
    return _mlp(gathered, W1, b1.reshape(1, -1), W2, b2.reshape(1, -1),
                W3, b3.reshape(1, -1))


# speedup vs baseline: 1.9068x; 1.9068x over previous
import functools

import jax
import jax.numpy as jnp
from jax import lax
from jax.experimental import pallas as pl
from jax.experimental.pallas import tpu as pltpu
from jax.experimental.pallas import tpu_sc as plsc

BATCH = 16384
EMB_D = 64
NROWS = 1000000
NC = 2
NS = 16
NW = NC * NS
L = 16

NSLAB = 7813
LAST_SLAB = 7812
RANGE = 245
NGRP = 16
GS = 16
MYCAP = 704
GCAP = 144
SENTPACK = 1 << 30


def _sc_gather_body(idx_hbm, tableT_hbm, out_hbm,
                    idx_v, my_pk, grp_pk, slab_v, rows_c, wpos_s,
                    ssem, wsem):
    wid = lax.axis_index("s") * NC + lax.axis_index("c")
    s0 = wid * RANGE
    iota = lax.iota(jnp.int32, L)
    sent_v = jnp.full((L,), SENTPACK, jnp.int32)

    for c in range(MYCAP // L):
        my_pk[pl.ds(c * L, L)] = sent_v

    lo = s0 * 128
    hi = (s0 + RANGE) * 128
    ICH = 2048

    cnt0 = jnp.int32(0)
    for ch in range(BATCH // ICH):
        pltpu.sync_copy(idx_hbm.at[pl.ds(ch * ICH, ICH)], idx_v)

        def l0(k, cnt, _ch=ch):
            r = idx_v[pl.ds(k * L, L)]
            ge = ((r - lo) >> 31) + 1
            lt = -((r - hi) >> 31)
            x = ge * lt
            m = x > 0
            pk = ((r - lo) << 14) | (iota + _ch * ICH + k * L)
            _, sv = plsc.sort_key_val(x, jnp.where(m, pk, SENTPACK),
                                      descending=True)
            my_pk[pl.ds(cnt, L)] = sv
            return cnt + plsc.all_reduce_population_count(m)[0]

        cnt0 = lax.fori_loop(0, ICH // L, l0, cnt0)

    def group_body(g, hitcnt):
        g0 = s0 + g * GS
        for c in range(GCAP // L):
            grp_pk[pl.ds(c * L, L)] = sent_v

        def l1(c, gcnt):
            pk = my_pk[pl.ds(c * L, L)]
            sl = pk >> 21
            ge = ((sl - g * GS) >> 31) + 1
            lt = -((sl - (g * GS + GS)) >> 31)
            x = ge * lt
            m = x > 0
            _, sv = plsc.sort_key_val(x, jnp.where(m, pk, SENTPACK),
                                      descending=True)
            grp_pk[pl.ds(gcnt, L)] = sv
            return gcnt + plsc.all_reduce_population_count(m)[0]

        lax.fori_loop(0, MYCAP // L, l1, jnp.int32(0))

        def fetch(s, b):
            pltpu.async_copy(tableT_hbm.at[:, pl.ds(s * 128, 128)],
                             slab_v.at[b], ssem)

        @pl.when(g0 <= LAST_SLAB)
        def _():
            fetch(g0, 0)

        def slab_body(t, hc):
            s = g0 + t

            @pl.when(s <= LAST_SLAB)
            def _():
                pltpu.make_async_copy(
                    tableT_hbm.at[:, pl.ds(s * 128, 128)],
                    slab_v.at[t % 2], ssem).wait()

            @pl.when(jnp.logical_and(t + 1 < GS, s + 1 <= LAST_SLAB))
            def _():
                fetch(s + 1, (t + 1) % 2)

            slab = slab_v.at[t % 2]
            for c in range(GCAP // L):
                pk = grp_pk[pl.ds(c * L, L)]
                d = (pk >> 21) - (s - s0)
                x = 1 - jnp.minimum(jnp.abs(d), 1)
                m = x > 0
                sk, sv = plsc.sort_key_val(x, jnp.where(m, pk, SENTPACK),
                                           descending=True)

                @pl.when(sk[0] > 0)
                def _():
                    for l in range(L):
                        @pl.when(sk[l] > 0)
                        def _():
                            e = sv[l]
                            lane = (e >> 14) & 127
                            k = hc + l
                            wpos_s[k] = e & 16383
                            lane_v = iota * 0 + lane
                            for c4 in range(EMB_D // L):
                                v = plsc.load_gather(
                                    slab, [iota + c4 * L, lane_v])
                                rows_c[k, pl.ds(c4 * L, L)] = v

                hc = hc + plsc.all_reduce_population_count(m)[0]
            return hc

        return lax.fori_loop(0, GS, slab_body, hitcnt)

    hitcnt = lax.fori_loop(0, NGRP, group_body, jnp.int32(0))

    def wr(k, _):
        pltpu.async_copy(rows_c.at[k], out_hbm.at[wpos_s[k]], wsem)
        return _

    lax.fori_loop(0, hitcnt, wr, 0)

    def drain(k, _):
        pltpu.make_async_copy(rows_c.at[0], out_hbm.at[0], wsem).wait()
        return _

    lax.fori_loop(0, hitcnt, drain, 0)


def _sc_gather(idx, tableT):
    mesh = plsc.VectorSubcoreMesh(core_axis_name="c", subcore_axis_name="s")
    k = functools.partial(
        pl.kernel,
        mesh=mesh,
        out_type=jax.ShapeDtypeStruct((BATCH, EMB_D), jnp.float32),
        scratch_types=[
            pltpu.VMEM((2048,), jnp.int32),
            pltpu.VMEM((MYCAP,), jnp.int32),
            pltpu.VMEM((GCAP,), jnp.int32),
            pltpu.VMEM((2, EMB_D, 128), jnp.float32),
            pltpu.VMEM((MYCAP, EMB_D), jnp.float32),
            pltpu.SMEM((MYCAP,), jnp.int32),
            pltpu.SemaphoreType.DMA,
            pltpu.SemaphoreType.DMA,
        ],
        compiler_params=pltpu.CompilerParams(disable_bounds_checks=True,
                                             needs_layout_passes=False),
    )(_sc_gather_body)
    return k(idx, tableT)


def _mlp_body(x_ref, w1_ref, b1_ref, w2_ref, b2_ref, w3_ref, b3_ref, o_ref):
    x = x_ref[...]
    h = jnp.dot(x, w1_ref[...], preferred_element_type=jnp.float32)
    h = jnp.maximum(h + b1_ref[...], 0.0)
    h = jnp.dot(h, w2_ref[...], preferred_element_type=jnp.float32)
    h = jnp.maximum(h + b2_ref[...], 0.0)
    y = jnp.dot(h, w3_ref[...], preferred_element_type=jnp.float32)
    y = y + b3_ref[...]
    norm = jnp.sqrt(jnp.sum(y * y, axis=1, keepdims=True))
    o_ref[...] = y / jnp.maximum(norm, 1e-12)


def _mlp(x, W1, b1, W2, b2, W3, b3):
    blk = 2048
    grid = (BATCH // blk,)
    full = lambda shape: pl.BlockSpec(shape, lambda i: (0, 0))
    return pl.pallas_call(
        _mlp_body,
        grid=grid,
        in_specs=[
            pl.BlockSpec((blk, EMB_D), lambda i: (i, 0)),
            full(W1.shape), full(b1.shape), full(W2.shape),
            full(b2.shape), full(W3.shape), full(b3.shape),
        ],
        out_specs=pl.BlockSpec((blk, EMB_D), lambda i: (i, 0)),
        out_shape=jax.ShapeDtypeStruct((BATCH, EMB_D), jnp.float32),
    )(x, W1, b1, W2, b2, W3, b3)


def kernel(user_ids, emb_table, W1, b1, W2, b2, W3, b3):
    idx = user_ids.astype(jnp.int32)
    gathered = _sc_gather(idx, emb_table.T)
    return _mlp(gathered, W1, b1.reshape(1, -1), W2, b2.reshape(1, -1),
                W3, b3.reshape(1, -1))


# speedup vs baseline: 3.6955x; 1.9380x over previous
import functools

import jax
import jax.numpy as jnp
from jax import lax
from jax.experimental import pallas as pl
from jax.experimental.pallas import tpu as pltpu
from jax.experimental.pallas import tpu_sc as plsc

BATCH = 16384
EMB_D = 64
NROWS = 1000000
NC = 2
NS = 16
NW = NC * NS
L = 16

NSLAB = 7813
LAST_SLAB = 7812
RANGE = 245
NGRP = 16
GS = 16
MYCAP = 704
GCAP = 144
SENTPACK = 1 << 30


def _sc_gather_body(idx_hbm, tableT_hbm, out_hbm,
                    idx_v, my_pk, grp_pk, slab_v, rows_c, wpos_s, cnt_s,
                    ssem, wsem):
    wid = lax.axis_index("s") * NC + lax.axis_index("c")
    s0 = wid * RANGE
    iota = lax.iota(jnp.int32, L)
    sent_v = jnp.full((L,), SENTPACK, jnp.int32)

    for c in range(MYCAP // L):
        my_pk[pl.ds(c * L, L)] = sent_v

    lo = s0 * 128
    hi = (s0 + RANGE) * 128
    ICH = 2048

    cnt0 = jnp.int32(0)
    for ch in range(BATCH // ICH):
        pltpu.sync_copy(idx_hbm.at[pl.ds(ch * ICH, ICH)], idx_v)

        def l0(k, cnt, _ch=ch):
            r = idx_v[pl.ds(k * L, L)]
            ge = ((r - lo) >> 31) + 1
            lt = -((r - hi) >> 31)
            x = ge * lt
            m = x > 0
            pk = ((r - lo) << 14) | (iota + _ch * ICH + k * L)
            _, sv = plsc.sort_key_val(x, jnp.where(m, pk, SENTPACK),
                                      descending=True)
            my_pk[pl.ds(cnt, L)] = sv
            return cnt + plsc.all_reduce_population_count(m)[0]

        cnt0 = lax.fori_loop(0, ICH // L, l0, cnt0)
    cnt_s[0] = cnt0

    def group_body(g, hitcnt):
        g0 = s0 + g * GS
        for c in range(GCAP // L):
            grp_pk[pl.ds(c * L, L)] = sent_v

        cnt_s[1] = 0

        def l1(c, carry):
            @pl.when(c * L < cnt_s[0])
            def _():
                gcnt = cnt_s[1]
                pk = my_pk[pl.ds(c * L, L)]
                sl = pk >> 21
                ge = ((sl - g * GS) >> 31) + 1
                lt = -((sl - (g * GS + GS)) >> 31)
                x = ge * lt
                m = x > 0
                _, sv = plsc.sort_key_val(x, jnp.where(m, pk, SENTPACK),
                                          descending=True)
                grp_pk[pl.ds(gcnt, L)] = sv
                cnt_s[1] = gcnt + plsc.all_reduce_population_count(m)[0]
            return carry

        lax.fori_loop(0, MYCAP // L, l1, 0)

        def fetch(s, b):
            pltpu.async_copy(tableT_hbm.at[:, pl.ds(s * 128, 128)],
                             slab_v.at[b], ssem)

        @pl.when(g0 <= LAST_SLAB)
        def _():
            fetch(g0, 0)

        def slab_body(t, hc):
            s = g0 + t

            @pl.when(s <= LAST_SLAB)
            def _():
                pltpu.make_async_copy(
                    tableT_hbm.at[:, pl.ds(s * 128, 128)],
                    slab_v.at[t % 2], ssem).wait()

            @pl.when(jnp.logical_and(t + 1 < GS, s + 1 <= LAST_SLAB))
            def _():
                fetch(s + 1, (t + 1) % 2)

            slab = slab_v.at[t % 2]
            for c in range(GCAP // L):
                @pl.when(c * L < cnt_s[1])
                def _():
                    hc = cnt_s[2]
                    pk = grp_pk[pl.ds(c * L, L)]
                    d = (pk >> 21) - (s - s0)
                    x = 1 - jnp.minimum(jnp.abs(d), 1)
                    m = x > 0
                    sk, sv = plsc.sort_key_val(x, jnp.where(m, pk, SENTPACK),
                                               descending=True)

                    @pl.when(sk[0] > 0)
                    def _():
                        for l in range(L):
                            @pl.when(sk[l] > 0)
                            def _():
                                e = sv[l]
                                lane = (e >> 14) & 127
                                k = hc + l
                                wpos_s[k] = e & 16383
                                lane_v = iota * 0 + lane
                                for c4 in range(EMB_D // L):
                                    v = plsc.load_gather(
                                        slab, [iota + c4 * L, lane_v])
                                    rows_c[k, pl.ds(c4 * L, L)] = v

                        cnt_s[2] = hc + \
                            plsc.all_reduce_population_count(m)[0]
            return hc

        return lax.fori_loop(0, GS, slab_body, hitcnt)

    cnt_s[2] = 0
    lax.fori_loop(0, NGRP, group_body, jnp.int32(0))
    hitcnt = cnt_s[2]

    def wr(k, _):
        pltpu.async_copy(rows_c.at[k], out_hbm.at[wpos_s[k]], wsem)
        return _

    lax.fori_loop(0, hitcnt, wr, 0)

    def drain(k, _):
        pltpu.make_async_copy(rows_c.at[0], out_hbm.at[0], wsem).wait()
        return _

    lax.fori_loop(0, hitcnt, drain, 0)


def _sc_gather(idx, tableT):
    mesh = plsc.VectorSubcoreMesh(core_axis_name="c", subcore_axis_name="s")
    k = functools.partial(
        pl.kernel,
        mesh=mesh,
        out_type=jax.ShapeDtypeStruct((BATCH, EMB_D), jnp.float32),
        scratch_types=[
            pltpu.VMEM((2048,), jnp.int32),
            pltpu.VMEM((MYCAP,), jnp.int32),
            pltpu.VMEM((GCAP,), jnp.int32),
            pltpu.VMEM((2, EMB_D, 128), jnp.float32),
            pltpu.VMEM((MYCAP, EMB_D), jnp.float32),
            pltpu.SMEM((MYCAP,), jnp.int32),
            pltpu.SMEM((8,), jnp.int32),
            pltpu.SemaphoreType.DMA,
            pltpu.SemaphoreType.DMA,
        ],
        compiler_params=pltpu.CompilerParams(disable_bounds_checks=True,
                                             needs_layout_passes=False),
    )(_sc_gather_body)
    return k(idx, tableT)


def _mlp_body(x_ref, w1_ref, b1_ref, w2_ref, b2_ref, w3_ref, b3_ref, o_ref):
    x = x_ref[...]
    h = jnp.dot(x, w1_ref[...], preferred_element_type=jnp.float32)
    h = jnp.maximum(h + b1_ref[...], 0.0)
    h = jnp.dot(h, w2_ref[...], preferred_element_type=jnp.float32)
    h = jnp.maximum(h + b2_ref[...], 0.0)
    y = jnp.dot(h, w3_ref[...], preferred_element_type=jnp.float32)
    y = y + b3_ref[...]
    norm = jnp.sqrt(jnp.sum(y * y, axis=1, keepdims=True))
    o_ref[...] = y / jnp.maximum(norm, 1e-12)


def _mlp(x, W1, b1, W2, b2, W3, b3):
    blk = 2048
    grid = (BATCH // blk,)
    full = lambda shape: pl.BlockSpec(shape, lambda i: (0, 0))
    return pl.pallas_call(
        _mlp_body,
        grid=grid,
        in_specs=[
            pl.BlockSpec((blk, EMB_D), lambda i: (i, 0)),
            full(W1.shape), full(b1.shape), full(W2.shape),
            full(b2.shape), full(W3.shape), full(b3.shape),
        ],
        out_specs=pl.BlockSpec((blk, EMB_D), lambda i: (i, 0)),
        out_shape=jax.ShapeDtypeStruct((BATCH, EMB_D), jnp.float32),
    )(x, W1, b1, W2, b2, W3, b3)


def kernel(user_ids, emb_table, W1, b1, W2, b2, W3, b3):
    idx = user_ids.astype(jnp.int32)
    gathered = _sc_gather(idx, emb_table.T)
    return _mlp(gathered, W1, b1.reshape(1, -1), W2, b2.reshape(1, -1),
                W3, b3.reshape(1, -1))
